# baseline (speedup 1.0000x reference)
import functools

import jax
import jax.numpy as jnp
from jax import lax
from jax.experimental import pallas as pl
from jax.experimental.pallas import tpu as pltpu
from jax.experimental.pallas import tpu_sc as plsc

EPS = 0.2
BLK = 256
PIECE = 64
NCHUNK = 2


def _router_body(E, x_ref, wt_ref, b_ref, gate_ref, cost_ref, cnt_ref):
    t = pl.program_id(0)
    logits = jnp.dot(x_ref[...], wt_ref[...],
                     preferred_element_type=jnp.float32) + b_ref[...]
    m = jnp.max(logits, axis=1, keepdims=True)
    ex = jnp.exp(logits - m)
    w = ex / jnp.sum(ex, axis=1, keepdims=True)
    cols = jax.lax.broadcasted_iota(jnp.int32, w.shape, 1)
    prefix_cols, rank_cols = [], []
    for j in range(E):
        wj = w[:, j:j + 1]
        ranked_above = (w > wj) | ((w == wj) & (cols < j))
        prefix_cols.append(
            jnp.sum(jnp.where(ranked_above, w, 0.0), axis=1, keepdims=True))
        rank_cols.append(
            jnp.sum(ranked_above.astype(jnp.int32), axis=1, keepdims=True))
    prefix = jnp.concatenate(prefix_cols, axis=1)
    rank = jnp.concatenate(rank_cols, axis=1)
    sw = jnp.maximum(0.0, jnp.minimum(w, (1.0 - EPS) - prefix))

    def perm_by_rank(v):
        outs = []
        for p in range(E):
            outs.append(jnp.sum(jnp.where(rank == p, v, 0.0), axis=1,
                                keepdims=True))
        return jnp.concatenate(outs, axis=1)

    gate = perm_by_rank(perm_by_rank(sw))
    gate_ref[...] = gate
    active = sw > 0.0
    num_active = jnp.sum(active.astype(jnp.float32), axis=1, keepdims=True)
    min_active = jnp.min(jnp.where(active, sw, jnp.inf), axis=1,
                         keepdims=True)
    cost_ref[...] = num_active - 1.0 + min_active

    blk_cnt = jnp.sum((gate > 0.0).astype(jnp.int32), axis=0, keepdims=True)

    @pl.when(t == 0)
    def _():
        cnt_ref[...] = blk_cnt

    @pl.when(t > 0)
    def _():
        cnt_ref[...] += blk_cnt


def _dispatch_body(N, D, E, M, gate_t_hbm, x_hbm, tok_hbm, sgate_hbm,
                   xs_hbm, bounds_hbm, grow_v, tok_v, sg_v, idx_v, rows_v,
                   bnd_v, sem):
    cid = lax.axis_index("c")
    sid = lax.axis_index("s")
    e = sid
    half = cid
    L = 16

    pltpu.sync_copy(gate_t_hbm.at[e], grow_v)

    lanes = lax.iota(jnp.int32, L)
    zeros_i = jnp.zeros((L,), jnp.int32)
    zeros_f = jnp.zeros((L,), jnp.float32)

    def zero_body(i, carry):
        tok_v[pl.ds(i * L, L)] = zeros_i
        sg_v[pl.ds(i * L, L)] = zeros_f
        return carry

    lax.fori_loop(0, N // L, zero_body, 0)

    def compress_body(c, ptr):
        g = grow_v[pl.ds(c * L, L)]
        mask = g > 0.0
        toks = c * L + lanes
        inc = plsc.cumsum(mask.astype(jnp.int32))
        pos = ptr + inc - 1
        plsc.store_scatter(tok_v, [pos], toks, mask=mask)
        plsc.store_scatter(sg_v, [pos], g, mask=mask)
        return ptr + jnp.max(inc)

    ptr = jnp.int32(0)
    bnd = jnp.zeros((L,), jnp.int32)
    for seg in range(NCHUNK):
        bnd = jnp.where(lanes == seg, ptr, bnd)
        ptr = lax.fori_loop(seg * (M // L), (seg + 1) * (M // L),
                            compress_body, ptr)
    bnd = jnp.where(lanes == NCHUNK, ptr, bnd)
    bnd_v[...] = bnd

    @pl.when(cid == 0)
    def _():
        pltpu.sync_copy(tok_v, tok_hbm.at[e])
        pltpu.sync_copy(sg_v, sgate_hbm.at[e])
        pltpu.sync_copy(bnd_v, bounds_hbm.at[e])

    cnt = ptr

    def gather_body(p, carry):
        @pl.when((lax.rem(p, 2) == half) & (p * PIECE < cnt))
        def _():
            idx_ref = tok_v.at[pl.ds(p * PIECE, PIECE)]
            pltpu.async_copy(x_hbm.at[idx_ref], rows_v, sem).wait()
            pltpu.sync_copy(rows_v, xs_hbm.at[e, pl.ds(p * PIECE, PIECE)])
        return carry

    lax.fori_loop(0, N // PIECE, gather_body, 0)


def _gmm_body(be_ref, br_ref, nl_ref, xs_ref, w_ref, b_ref, g_ref, y_ref):
    b = pl.program_id(0)

    @pl.when(b < nl_ref[0])
    def _():
        y = jnp.dot(xs_ref[0], w_ref[0],
                    preferred_element_type=jnp.float32) + b_ref[0]
        y_ref[0] = g_ref[0] * y


def _combine_body(N, OUT, E, M, y_hbm, tok_hbm, bounds_hbm, out_hbm,
                  bnd_v, idx_v, idx2_v, yrows_v, orows_v, z_v, sem):
    cid = lax.axis_index("c")
    sid = lax.axis_index("s")
    e = sid
    L = 16
    TRASH = N

    def zrow(p, c):
        for q in range(OUT // L):
            z_v[p, pl.ds(q * L, L)] = jnp.zeros((L,), jnp.float32)
        return c

    lax.fori_loop(0, 8, zrow, 0)

    pltpu.sync_copy(bounds_hbm, bnd_v)
    lanes = lax.iota(jnp.int32, L)

    wr = M // E
    for q in range(wr // 8):
        pltpu.sync_copy(
            z_v, out_hbm.at[pl.ds(cid * M + sid * wr + q * 8, 8)])
    plsc.subcore_barrier()

    for e2 in range(E):
        brow = bnd_v[pl.ds(e2 * 16, L)]
        s0 = jnp.sum(jnp.where(lanes == cid, brow, 0))
        s1 = jnp.sum(jnp.where(lanes == cid + 1, brow, 0))
        s0a = pl.multiple_of((s0 // 8) * 8, 8)
        head = s0 - s0a
        npairs = s1 - s0a

        def piece_body(p2, carry):
            p = p2 * E + sid

            @pl.when(p * PIECE < npairs)
            def _():
                off = pl.multiple_of(e2 * N + s0a + p * PIECE, 8)
                pltpu.sync_copy(tok_hbm.at[pl.ds(off, PIECE)], idx_v)
                pltpu.sync_copy(y_hbm.at[pl.ds(off, PIECE)], yrows_v)
                for cch in range(PIECE // L):
                    iv = idx_v[pl.ds(cch * L, L)]
                    k = p * PIECE + cch * L + lanes
                    valid = (k >= head) & (k < npairs)
                    iv = jnp.where(valid, iv, TRASH)
                    idx2_v[pl.ds(cch * L, L)] = iv
                pltpu.async_copy(out_hbm.at[idx2_v], orows_v, sem).wait()

                def add_row(r, c2):
                    for c in range(OUT // L):
                        sl = pl.ds(c * L, L)
                        orows_v[r, sl] = orows_v[r, sl] + yrows_v[r, sl]
                    return c2

                lax.fori_loop(0, PIECE, add_row, 0)
                pltpu.async_copy(orows_v, out_hbm.at[idx2_v], sem).wait()
            return carry

        lax.fori_loop(0, N // PIECE // E + 1, piece_body, 0)
        plsc.subcore_barrier()


def kernel(x, Wsel, bsel, Wexp, bexp):
    N, D = x.shape
    E, _, OUT = Wexp.shape
    M = N // NCHUNK

    tb_r = min(1024, N)
    gate, cost, counts = pl.pallas_call(
        functools.partial(_router_body, E),
        grid=(N // tb_r,),
        in_specs=[
            pl.BlockSpec((tb_r, D), lambda t: (t, 0)),
            pl.BlockSpec((D, E), lambda t: (0, 0)),
            pl.BlockSpec((1, E), lambda t: (0, 0)),
        ],
        out_specs=[
            pl.BlockSpec((tb_r, E), lambda t: (t, 0)),
            pl.BlockSpec((tb_r, 1), lambda t: (t, 0)),
            pl.BlockSpec((1, E), lambda t: (0, 0)),
        ],
        out_shape=[
            jax.ShapeDtypeStruct((N, E), jnp.float32),
            jax.ShapeDtypeStruct((N, 1), jnp.float32),
            jax.ShapeDtypeStruct((1, E), jnp.int32),
        ],
    )(x, Wsel.T, bsel.reshape(1, E))

    gate_t = gate.T

    slot_tok, slot_gate, xs, bounds = _dispatch_call(gate_t, x, N, D, E, M)

    cnt = counts[0]
    nb = (cnt + BLK - 1) // BLK
    off = jnp.concatenate([jnp.zeros((1,), jnp.int32),
                           jnp.cumsum(nb, dtype=jnp.int32)])
    nlive = off[E]
    NBLK = E * (N // BLK)
    b_idx = jnp.arange(NBLK, dtype=jnp.int32)
    be = jnp.sum((b_idx[:, None] >= off[None, 1:]).astype(jnp.int32), axis=1)
    be = jnp.minimum(be, E - 1)
    last_b = jnp.maximum(nlive - 1, 0)
    last_e = be[last_b]
    be = jnp.where(b_idx < nlive, be, last_e)
    br = jnp.where(b_idx < nlive, b_idx - off[be], last_b - off[last_e])
    nl = jnp.reshape(nlive, (1,))

    grid_spec = pltpu.PrefetchScalarGridSpec(
        num_scalar_prefetch=3,
        grid=(NBLK,),
        in_specs=[
            pl.BlockSpec((1, BLK, D), lambda b, be, br, nl: (be[b], br[b], 0)),
            pl.BlockSpec((1, D, OUT), lambda b, be, br, nl: (be[b], 0, 0)),
            pl.BlockSpec((1, 1, OUT), lambda b, be, br, nl: (be[b], 0, 0)),
            pl.BlockSpec((1, BLK, 1), lambda b, be, br, nl: (be[b], br[b], 0)),
        ],
        out_specs=pl.BlockSpec((1, BLK, OUT),
                               lambda b, be, br, nl: (be[b], br[b], 0)),
    )
    y = pl.pallas_call(
        _gmm_body,
        grid_spec=grid_spec,
        out_shape=jax.ShapeDtypeStruct((E, N, OUT), jnp.float32),
    )(be, br, nl, xs, Wexp, bexp.reshape(E, 1, OUT),
      slot_gate.reshape(E, N, 1))

    out = _combine_call(y, slot_tok, bounds, N, OUT, E, M)

    return out, cost.reshape(N)


def _sc_mesh():
    return plsc.VectorSubcoreMesh(core_axis_name="c", subcore_axis_name="s",
                                  num_cores=2, num_subcores=16)


def _dispatch_call(gate_t, x, N, D, E, M):
    return pl.kernel(
        functools.partial(_dispatch_body, N, D, E, M),
        out_type=[
            jax.ShapeDtypeStruct((E, N), jnp.int32),
            jax.ShapeDtypeStruct((E, N), jnp.float32),
            jax.ShapeDtypeStruct((E, N, D), jnp.float32),
            jax.ShapeDtypeStruct((E, 16), jnp.int32),
        ],
        mesh=_sc_mesh(),
        compiler_params=pltpu.CompilerParams(needs_layout_passes=False),
        scratch_types=[
            pltpu.VMEM((N,), jnp.float32),
            pltpu.VMEM((N,), jnp.int32),
            pltpu.VMEM((N,), jnp.float32),
            pltpu.VMEM((PIECE,), jnp.int32),
            pltpu.VMEM((PIECE, D), jnp.float32),
            pltpu.VMEM((16,), jnp.int32),
            pltpu.SemaphoreType.DMA,
        ],
    )(gate_t, x)


def _combine_call(y, slot_tok, bounds, N, OUT, E, M):
    out_padded = pl.kernel(
        functools.partial(_combine_body, N, OUT, E, M),
        out_type=jax.ShapeDtypeStruct((N + 8, OUT), jnp.float32),
        mesh=_sc_mesh(),
        name="smoe_combine",
        compiler_params=pltpu.CompilerParams(needs_layout_passes=False),
        scratch_types=[
            pltpu.VMEM((E * 16,), jnp.int32),
            pltpu.VMEM((PIECE,), jnp.int32),
            pltpu.VMEM((PIECE,), jnp.int32),
            pltpu.VMEM((PIECE, OUT), jnp.float32),
            pltpu.VMEM((PIECE, OUT), jnp.float32),
            pltpu.VMEM((8, OUT), jnp.float32),
            pltpu.SemaphoreType.DMA,
        ],
    )(y.reshape(E * N, OUT), slot_tok.reshape(E * N), bounds.reshape(E * 16))
    return out_padded[:N]


# speedup vs baseline: 1.8007x; 1.8007x over previous
import functools

import jax
import jax.numpy as jnp
from jax.experimental import pallas as pl

EPS = 0.2


def _router_body(E, x_ref, wt_ref, b_ref, gate_ref, cost_ref):
    logits = jnp.dot(x_ref[...], wt_ref[...],
                     preferred_element_type=jnp.float32) + b_ref[...]
    m = jnp.max(logits, axis=1, keepdims=True)
    ex = jnp.exp(logits - m)
    w = ex / jnp.sum(ex, axis=1, keepdims=True)
    cols = jax.lax.broadcasted_iota(jnp.int32, w.shape, 1)
    prefix_cols = []
    rank_cols = []
    for j in range(E):
        wj = w[:, j:j + 1]
        ranked_above = (w > wj) | ((w == wj) & (cols < j))
        prefix_cols.append(
            jnp.sum(jnp.where(ranked_above, w, 0.0), axis=1, keepdims=True))
        rank_cols.append(
            jnp.sum(ranked_above.astype(jnp.int32), axis=1, keepdims=True))
    prefix = jnp.concatenate(prefix_cols, axis=1)
    rank = jnp.concatenate(rank_cols, axis=1)
    sw = jnp.maximum(0.0, jnp.minimum(w, (1.0 - EPS) - prefix))

    def perm_by_rank(v):
        outs = []
        for p in range(E):
            outs.append(jnp.sum(jnp.where(rank == p, v, 0.0), axis=1,
                                keepdims=True))
        return jnp.concatenate(outs, axis=1)

    gate_ref[...] = perm_by_rank(perm_by_rank(sw))
    active = sw > 0.0
    num_active = jnp.sum(active.astype(jnp.float32), axis=1, keepdims=True)
    min_active = jnp.min(jnp.where(active, sw, jnp.inf), axis=1,
                         keepdims=True)
    cost_ref[...] = num_active - 1.0 + min_active


def _expert_body(E, x_ref, w_ref, b_ref, g_ref, out_ref):
    e = pl.program_id(1)
    cols = jax.lax.broadcasted_iota(jnp.int32, g_ref.shape, 1)
    ge = jnp.sum(jnp.where(cols == e, g_ref[...], 0.0), axis=1, keepdims=True)
    y = jnp.dot(x_ref[...], w_ref[0],
                preferred_element_type=jnp.float32) + b_ref[0]
    contrib = ge * y

    @pl.when(e == 0)
    def _():
        out_ref[...] = contrib

    @pl.when(e > 0)
    def _():
        out_ref[...] += contrib


def kernel(x, Wsel, bsel, Wexp, bexp):
    N, D = x.shape
    E, _, OUT = Wexp.shape

    tb_r = min(1024, N)
    gate, cost = pl.pallas_call(
        functools.partial(_router_body, E),
        grid=(N // tb_r,),
        in_specs=[
            pl.BlockSpec((tb_r, D), lambda t: (t, 0)),
            pl.BlockSpec((D, E), lambda t: (0, 0)),
            pl.BlockSpec((1, E), lambda t: (0, 0)),
        ],
        out_specs=[
            pl.BlockSpec((tb_r, E), lambda t: (t, 0)),
            pl.BlockSpec((tb_r, 1), lambda t: (t, 0)),
        ],
        out_shape=[
            jax.ShapeDtypeStruct((N, E), jnp.float32),
            jax.ShapeDtypeStruct((N, 1), jnp.float32),
        ],
    )(x, Wsel.T, bsel.reshape(1, E))

    tb = min(2048, N)
    out = pl.pallas_call(
        functools.partial(_expert_body, E),
        grid=(N // tb, E),
        in_specs=[
            pl.BlockSpec((tb, D), lambda t, e: (t, 0)),
            pl.BlockSpec((1, D, OUT), lambda t, e: (e, 0, 0)),
            pl.BlockSpec((1, 1, OUT), lambda t, e: (e, 0, 0)),
            pl.BlockSpec((tb, E), lambda t, e: (t, 0)),
        ],
        out_specs=pl.BlockSpec((tb, OUT), lambda t, e: (t, 0)),
        out_shape=jax.ShapeDtypeStruct((N, OUT), jnp.float32),
    )(x, Wexp, bexp.reshape(E, 1, OUT), gate)

    return out, cost.reshape(N)


# speedup vs baseline: 1.8202x; 1.0108x over previous
import functools

import jax
import jax.numpy as jnp
from jax.experimental import pallas as pl
from jax.experimental import pallas as _pl
from jax.experimental.pallas import tpu as pltpu

EPS = 0.2


def _body(E, x_ref, wt_ref, b_ref, w_ref, be_ref, out_ref, cost_ref, g_ref):
    e = pl.program_id(1)

    @pl.when(e == 0)
    def _():
        logits = jnp.dot(x_ref[...], wt_ref[...],
                         preferred_element_type=jnp.float32) + b_ref[...]
        m = jnp.max(logits, axis=1, keepdims=True)
        ex = jnp.exp(logits - m)
        w = ex / jnp.sum(ex, axis=1, keepdims=True)
        cols = jax.lax.broadcasted_iota(jnp.int32, w.shape, 1)
        prefix_cols, rank_cols = [], []
        for j in range(E):
            wj = w[:, j:j + 1]
            ranked_above = (w > wj) | ((w == wj) & (cols < j))
            prefix_cols.append(
                jnp.sum(jnp.where(ranked_above, w, 0.0), axis=1,
                        keepdims=True))
            rank_cols.append(
                jnp.sum(ranked_above.astype(jnp.int32), axis=1,
                        keepdims=True))
        prefix = jnp.concatenate(prefix_cols, axis=1)
        rank = jnp.concatenate(rank_cols, axis=1)
        sw = jnp.maximum(0.0, jnp.minimum(w, (1.0 - EPS) - prefix))

        def perm_by_rank(v):
            outs = []
            for p in range(E):
                outs.append(jnp.sum(jnp.where(rank == p, v, 0.0), axis=1,
                                    keepdims=True))
            return jnp.concatenate(outs, axis=1)

        g_ref[...] = perm_by_rank(perm_by_rank(sw))
        active = sw > 0.0
        num_active = jnp.sum(active.astype(jnp.float32), axis=1,
                             keepdims=True)
        min_active = jnp.min(jnp.where(active, sw, jnp.inf), axis=1,
                             keepdims=True)
        cost_ref[...] = num_active - 1.0 + min_active

    cols = jax.lax.broadcasted_iota(jnp.int32, g_ref.shape, 1)
    ge = jnp.sum(jnp.where(cols == e, g_ref[...], 0.0), axis=1, keepdims=True)
    y = jnp.dot(x_ref[...], w_ref[0],
                preferred_element_type=jnp.float32) + be_ref[0]
    contrib = ge * y

    @pl.when(e == 0)
    def _():
        out_ref[...] = contrib

    @pl.when(e > 0)
    def _():
        out_ref[...] += contrib


def kernel(x, Wsel, bsel, Wexp, bexp):
    N, D = x.shape
    E, _, OUT = Wexp.shape

    tb = min(2048, N)
    out, cost = pl.pallas_call(
        functools.partial(_body, E),
        grid=(N // tb, E),
        in_specs=[
            pl.BlockSpec((tb, D), lambda t, e: (t, 0)),
            pl.BlockSpec((D, E), lambda t, e: (0, 0)),
            pl.BlockSpec((1, E), lambda t, e: (0, 0)),
            pl.BlockSpec((1, D, OUT), lambda t, e: (e, 0, 0)),
            pl.BlockSpec((1, 1, OUT), lambda t, e: (e, 0, 0)),
        ],
        out_specs=[
            pl.BlockSpec((tb, OUT), lambda t, e: (t, 0)),
            pl.BlockSpec((tb, 1), lambda t, e: (t, 0)),
        ],
        out_shape=[
            jax.ShapeDtypeStruct((N, OUT), jnp.float32),
            jax.ShapeDtypeStruct((N, 1), jnp.float32),
        ],
        scratch_shapes=[pltpu.VMEM((tb, E), jnp.float32)],
    )(x, Wsel.T, bsel.reshape(1, E), Wexp, bexp.reshape(E, 1, OUT))

    return out, cost.reshape(N)
